# baseline (speedup 1.0000x reference)
import jax
import jax.numpy as jnp
from jax import lax
from jax.experimental import pallas as pl
from jax.experimental.pallas import tpu as pltpu
from jax.experimental.pallas import tpu_sc as plsc

_K = 1024
_D = 256
_N = 16384
_R = 256
_G = _N // _R
_BETA = 0.25

_NC = 2
_NS = 16
_NW = _NC * _NS
_RPW = _N // _NW
_CH = 128
_LANES = 16


def _tc_body(zn_ref, en_ref, zf_ref, e_ref, oh_ref, idx_ref, perp_ref, counts):
    i = pl.program_id(0)
    zf = zf_ref[...]
    e = e_ref[...]
    mm = lax.dot_general(zf, e, (((1,), (1,)), ((), ())),
                         preferred_element_type=jnp.float32)
    d = (zn_ref[...] + en_ref[...]) - 2.0 * mm
    vmin = jnp.min(d, axis=1, keepdims=True)
    ks = lax.broadcasted_iota(jnp.int32, (_R, _K), 1)
    idx = jnp.min(jnp.where(d == vmin, ks, _K), axis=1)
    oh = (ks == idx[:, None]).astype(jnp.float32)
    oh_ref[...] = oh
    idx_ref[0, 0, :] = idx
    cnt = jnp.sum(oh, axis=0, keepdims=True)

    @pl.when(i == 0)
    def _init():
        counts[...] = cnt

    @pl.when(i > 0)
    def _acc():
        counts[...] = counts[...] + cnt

    @pl.when(i == _G - 1)
    def _fin():
        p = counts[...] * (1.0 / _N)
        ent = jnp.sum(p * jnp.log(p + 1e-10))
        perp_ref[...] = jnp.exp(-ent).reshape(1, 1)


def _tc_call(zn, en2, zf, emb):
    return pl.pallas_call(
        _tc_body,
        grid=(_G,),
        in_specs=[
            pl.BlockSpec((_R, 1), lambda i: (i, 0)),
            pl.BlockSpec((1, _K), lambda i: (0, 0)),
            pl.BlockSpec((_R, _D), lambda i: (i, 0)),
            pl.BlockSpec((_K, _D), lambda i: (0, 0)),
        ],
        out_specs=[
            pl.BlockSpec((_R, _K), lambda i: (i, 0)),
            pl.BlockSpec((1, 1, _R), lambda i: (i, 0, 0)),
            pl.BlockSpec((1, 1), lambda i: (0, 0)),
        ],
        out_shape=[
            jax.ShapeDtypeStruct((_N, _K), jnp.float32),
            jax.ShapeDtypeStruct((_G, 1, _R), jnp.int32),
            jax.ShapeDtypeStruct((1, 1), jnp.float32),
        ],
        scratch_shapes=[pltpu.VMEM((1, _K), jnp.float32)],
    )(zn, en2, zf, emb)


def _sc_body(idx_hbm, zf_hbm, emb_hbm, zq_out, part_out,
             idx_v, rows_v, zp_v, acc_v, sem):
    wid = lax.axis_index("s") * _NC + lax.axis_index("c")
    base = wid * _RPW
    acc = jnp.zeros((_LANES,), jnp.float32)
    for ch in range(_RPW // _CH):
        b = base + ch * _CH
        pltpu.sync_copy(idx_hbm.at[pl.ds(b, _CH)], idx_v)
        gather = pltpu.async_copy(emb_hbm.at[idx_v], rows_v, sem)
        pltpu.sync_copy(zf_hbm.at[pl.ds(b, _CH)], zp_v)
        gather.wait()

        def _row(r, acc):
            for c in range(_D // _LANES):
                sl = pl.ds(c * _LANES, _LANES)
                zq = rows_v[r, sl]
                zp = zp_v[r, sl]
                diff = zq - zp
                rows_v[r, sl] = zp + diff
                acc = acc + diff * diff
            return acc

        acc = lax.fori_loop(0, _CH, _row, acc)
        pltpu.sync_copy(rows_v, zq_out.at[pl.ds(b, _CH)])
    acc_v[...] = acc
    pltpu.sync_copy(acc_v, part_out.at[wid])


def _sc_call(idx_flat, zf, emb):
    mesh = plsc.VectorSubcoreMesh(core_axis_name="c", subcore_axis_name="s")
    kern = pl.kernel(
        _sc_body,
        mesh=mesh,
        out_type=[
            jax.ShapeDtypeStruct((_N, _D), jnp.float32),
            jax.ShapeDtypeStruct((_NW, _LANES), jnp.float32),
        ],
        scratch_types=[
            pltpu.VMEM((_CH,), jnp.int32),
            pltpu.VMEM((_CH, _D), jnp.float32),
            pltpu.VMEM((_CH, _D), jnp.float32),
            pltpu.VMEM((_LANES,), jnp.float32),
            pltpu.SemaphoreType.DMA,
        ],
    )
    return kern(idx_flat, zf, emb)


def kernel(z, embedding):
    B, D, H, W = z.shape
    zp = jnp.transpose(z, (0, 2, 3, 1))
    zf = zp.reshape(-1, D)
    zn = jnp.sum(zf ** 2, axis=1, keepdims=True)
    en = jnp.sum(embedding ** 2, axis=1)
    oh, idx3, perp2 = _tc_call(zn, en.reshape(1, _K), zf, embedding)
    idx_flat = idx3.reshape(_N)
    zq_st, parts = _sc_call(idx_flat, zf, embedding)
    m = jnp.sum(parts) * (1.0 / (_N * _D))
    loss = m + _BETA * m
    z_q_out = jnp.transpose(zq_st.reshape(B, H, W, D), (0, 3, 1, 2))
    return (loss, z_q_out, perp2[0, 0], oh, idx3.reshape(B, H, W))


# speedup vs baseline: 1.1883x; 1.1883x over previous
import jax
import jax.numpy as jnp
from jax import lax
from jax.experimental import pallas as pl
from jax.experimental.pallas import tpu as pltpu
from jax.experimental.pallas import tpu_sc as plsc

_K = 1024
_D = 256
_N = 16384
_R = 256
_G = _N // _R
_BETA = 0.25

_NC = 2
_NS = 16
_NW = _NC * _NS
_RPW = _N // _NW
_CH = 128
_LANES = 16


def _tc_body(en_ref, zf_ref, e_ref, oh_ref, idx_ref, perp_ref, counts):
    i = pl.program_id(0)
    zf = zf_ref[...]
    e = e_ref[...]
    mm = lax.dot_general(zf, e, (((1,), (1,)), ((), ())),
                         preferred_element_type=jnp.float32)
    zn = jnp.sum(zf * zf, axis=1, keepdims=True)
    d = (zn + en_ref[...]) - 2.0 * mm
    vmin = jnp.min(d, axis=1, keepdims=True)
    ksf = lax.broadcasted_iota(jnp.int32, (_R, _K), 1).astype(jnp.float32)
    idxf = jnp.min(jnp.where(d == vmin, ksf, 65536.0), axis=1, keepdims=True)
    oh = jnp.where(ksf == idxf, 1.0, 0.0)
    oh_ref[...] = oh
    idx_ref[0, 0, :] = idxf[:, 0].astype(jnp.int32)
    cnt = jnp.sum(oh, axis=0, keepdims=True)

    @pl.when(i == 0)
    def _init():
        counts[...] = cnt

    @pl.when(i > 0)
    def _acc():
        counts[...] = counts[...] + cnt

    @pl.when(i == _G - 1)
    def _fin():
        p = counts[...] * (1.0 / _N)
        ent = jnp.sum(p * jnp.log(p + 1e-10))
        perp_ref[...] = jnp.exp(-ent).reshape(1, 1)


def _tc_call(en2, zf, emb):
    return pl.pallas_call(
        _tc_body,
        grid=(_G,),
        in_specs=[
            pl.BlockSpec((1, _K), lambda i: (0, 0)),
            pl.BlockSpec((_R, _D), lambda i: (i, 0)),
            pl.BlockSpec((_K, _D), lambda i: (0, 0)),
        ],
        out_specs=[
            pl.BlockSpec((_R, _K), lambda i: (i, 0)),
            pl.BlockSpec((1, 1, _R), lambda i: (i, 0, 0)),
            pl.BlockSpec((1, 1), lambda i: (0, 0)),
        ],
        out_shape=[
            jax.ShapeDtypeStruct((_N, _K), jnp.float32),
            jax.ShapeDtypeStruct((_G, 1, _R), jnp.int32),
            jax.ShapeDtypeStruct((1, 1), jnp.float32),
        ],
        scratch_shapes=[pltpu.VMEM((1, _K), jnp.float32)],
    )(en2, zf, emb)


def _sc_body(idx_hbm, zf_hbm, emb_hbm, zq_out, part_out,
             idx_v, rows_v, zp_v, acc_v, sem):
    wid = lax.axis_index("s") * _NC + lax.axis_index("c")
    base = wid * _RPW
    acc = jnp.zeros((_LANES,), jnp.float32)
    for ch in range(_RPW // _CH):
        b = base + ch * _CH
        pltpu.sync_copy(idx_hbm.at[pl.ds(b, _CH)], idx_v)
        gather = pltpu.async_copy(emb_hbm.at[idx_v], rows_v, sem)
        pltpu.sync_copy(zf_hbm.at[pl.ds(b, _CH)], zp_v)
        gather.wait()

        def _row(r, acc):
            for c in range(_D // _LANES):
                sl = pl.ds(c * _LANES, _LANES)
                zq = rows_v[r, sl]
                zp = zp_v[r, sl]
                diff = zq - zp
                rows_v[r, sl] = zp + diff
                acc = acc + diff * diff
            return acc

        acc = lax.fori_loop(0, _CH, _row, acc)
        pltpu.sync_copy(rows_v, zq_out.at[pl.ds(b, _CH)])
    acc_v[...] = acc
    pltpu.sync_copy(acc_v, part_out.at[wid])


def _sc_call(idx_flat, zf, emb):
    mesh = plsc.VectorSubcoreMesh(core_axis_name="c", subcore_axis_name="s")
    kern = pl.kernel(
        _sc_body,
        mesh=mesh,
        out_type=[
            jax.ShapeDtypeStruct((_N, _D), jnp.float32),
            jax.ShapeDtypeStruct((_NW, _LANES), jnp.float32),
        ],
        scratch_types=[
            pltpu.VMEM((_CH,), jnp.int32),
            pltpu.VMEM((_CH, _D), jnp.float32),
            pltpu.VMEM((_CH, _D), jnp.float32),
            pltpu.VMEM((_LANES,), jnp.float32),
            pltpu.SemaphoreType.DMA,
        ],
    )
    return kern(idx_flat, zf, emb)


def kernel(z, embedding):
    B, D, H, W = z.shape
    zp = jnp.transpose(z, (0, 2, 3, 1))
    zf = zp.reshape(-1, D)
    en = jnp.sum(embedding ** 2, axis=1)
    oh, idx3, perp2 = _tc_call(en.reshape(1, _K), zf, embedding)
    idx_flat = idx3.reshape(_N)
    zq_st, parts = _sc_call(idx_flat, zf, embedding)
    m = jnp.sum(parts) * (1.0 / (_N * _D))
    loss = m + _BETA * m
    z_q_out = jnp.transpose(zq_st.reshape(B, H, W, D), (0, 3, 1, 2))
    return (loss, z_q_out, perp2[0, 0], oh, idx3.reshape(B, H, W))
